# baseline (speedup 1.0000x reference)
import jax
import jax.numpy as jnp
import numpy as np
from jax import lax
from jax.experimental import pallas as pl
from jax.experimental.pallas import tpu as pltpu
from jax.experimental.pallas import tpu_sc as plsc

N_NODES = 10000
N_EDGES = 320000
D_IN = 128
D_HID = 128
D_OUT = 16

NC = 2
NS = 16
NW = NC * NS
CHUNK = 128
EPAD = 327680
EPT = EPAD // NW
KCH = EPT // CHUNK
KQ = 40
NQ = KCH // KQ
NPAD = 10240
RPT = NPAD // NS
ZCH = 40
NZ = RPT // ZCH

_MESH = plsc.VectorSubcoreMesh(core_axis_name="c", subcore_axis_name="s")

_NPADE = EPAD - N_EDGES
_PAD_SRC = np.arange(_NPADE, dtype=np.int32) % N_NODES
_PAD_DST = (N_NODES + np.arange(_NPADE, dtype=np.int32) % (NPAD - N_NODES)).astype(np.int32)


def _zero_rows(ref, nrows, width):
  zf = jnp.zeros((16,), jnp.float32)

  def zrow(r, carry):
    for j in range(width // 16):
      ref[r, pl.ds(16 * j, 16)] = zf
    return carry

  lax.fori_loop(0, nrows, zrow, 0)


def _seg_sum_sc(width, tc_tiling=True, kq=40, nbuf=2, chunk=CHUNK):
  kch = EPT // chunk
  nq = kch // kq

  def body(g_hbm, src_hbm, dst_hbm, agg_hbm, src_q, dst_q, *rest):
    bufs = rest[:nbuf]
    zbuf = rest[nbuf]
    acc = rest[nbuf + 1]
    sems = rest[nbuf + 2:]
    c = lax.axis_index("c")
    s = lax.axis_index("s")
    wid = c * NS + s
    row0 = s * RPT

    _zero_rows(zbuf, ZCH, width)
    for t in range(NZ):
      pltpu.sync_copy(zbuf, acc.at[pl.ds(row0 + t * ZCH, ZCH)])

    plsc.subcore_barrier()

    for q in range(nq):
      pltpu.sync_copy(src_hbm.at[wid, pl.ds(q * kq, kq)], src_q)
      pltpu.sync_copy(dst_hbm.at[wid, pl.ds(q * kq, kq)], dst_q)
      for i in range(nbuf - 1):
        pltpu.async_copy(g_hbm.at[src_q.at[i]], bufs[i], sems[i])

      def group(p, carry):
        for i in range(nbuf):
          k = nbuf * p + i
          nxt = k + nbuf - 1
          b = i
          bn = (i + nbuf - 1) % nbuf

          @pl.when(nxt < kq)
          def _(nxt=nxt, bn=bn):
            pltpu.async_copy(g_hbm.at[src_q.at[nxt]], bufs[bn], sems[bn])

          pltpu.make_async_copy(g_hbm.at[src_q.at[k]], bufs[b],
                                sems[b]).wait()
          pltpu.sync_copy(bufs[b], acc.at[dst_q.at[k]], add=True)
        return carry

      lax.fori_loop(0, kq // nbuf, group, 0)

    plsc.subcore_barrier()

    pltpu.sync_copy(acc.at[pl.ds(row0, RPT)], agg_hbm.at[c, pl.ds(row0, RPT)])

  return pl.kernel(
      body,
      out_type=[jax.ShapeDtypeStruct((NC, NPAD, width), jnp.float32)],
      mesh=_MESH,
      compiler_params=pltpu.CompilerParams(use_tc_tiling_on_sc=tc_tiling),
      scratch_types=(
          [pltpu.VMEM((kq, chunk), jnp.int32),
           pltpu.VMEM((kq, chunk), jnp.int32)]
          + [pltpu.VMEM((chunk, width), jnp.float32)] * nbuf
          + [pltpu.VMEM((ZCH, width), jnp.float32),
             pltpu.VMEM_SHARED((NPAD, width), jnp.float32)]
          + [pltpu.SemaphoreType.DMA] * nbuf),
  )


def _count_sc():
  kq = 80

  def body(dst_hbm, cnt_hbm, dst_q, ones_v, zcnt, cacc, sem):
    c = lax.axis_index("c")
    s = lax.axis_index("s")
    wid = c * NS + s
    row0 = s * RPT

    _zero_rows(zcnt, ZCH, 16)
    for t in range(NZ):
      pltpu.sync_copy(zcnt, cacc.at[pl.ds(row0 + t * ZCH, ZCH)])

    onef = jnp.ones((16,), jnp.float32)

    def orow(r, carry):
      ones_v[r, :] = onef
      return carry

    lax.fori_loop(0, CHUNK, orow, 0)

    plsc.subcore_barrier()

    pltpu.sync_copy(dst_hbm.at[wid], dst_q)

    def group(p, carry):
      for i in range(8):
        pltpu.async_copy(ones_v, cacc.at[dst_q.at[8 * p + i]], sem,
                         add=True)
      for i in range(8):
        pltpu.make_async_copy(ones_v, cacc.at[dst_q.at[8 * p + i]],
                              sem).wait()
      return carry

    lax.fori_loop(0, kq // 8, group, 0)

    plsc.subcore_barrier()

    pltpu.sync_copy(cacc.at[pl.ds(row0, RPT)], cnt_hbm.at[c, pl.ds(row0, RPT)])

  return pl.kernel(
      body,
      out_type=[jax.ShapeDtypeStruct((NC, NPAD, 16), jnp.float32)],
      mesh=_MESH,
      compiler_params=pltpu.CompilerParams(use_tc_tiling_on_sc=False),
      scratch_types=[
          pltpu.VMEM((kq, CHUNK), jnp.int32),
          pltpu.VMEM((CHUNK, 16), jnp.float32),
          pltpu.VMEM((ZCH, 16), jnp.float32),
          pltpu.VMEM_SHARED((NPAD, 16), jnp.float32),
          pltpu.SemaphoreType.DMA,
      ],
  )


_seg128 = _seg_sum_sc(D_HID, tc_tiling=True, kq=40, nbuf=4, chunk=64)

def _seg_sum_spmem():
  chunk = 64
  kq = 40
  nbuf = 4
  cpt = EPAD // NS // chunk
  nq = cpt // kq

  def body(glo_hbm, ghi_hbm, src_hbm, dst_hbm, agg_hbm, src_q, dst_q,
           b0, b1, b2, b3, zbuf, table, acc, s0, s1, s2, s3):
    bufs = (b0, b1, b2, b3)
    sems = (s0, s1, s2, s3)
    c = lax.axis_index("c")
    s = lax.axis_index("s")
    row0 = s * RPT

    _zero_rows(zbuf, ZCH, chunk)
    for t in range(RPT // ZCH):
      pltpu.sync_copy(zbuf, acc.at[pl.ds(row0 + t * ZCH, ZCH)])

    @pl.when(jnp.logical_and(c == 0, s < NS - 1))
    def _():
      pltpu.sync_copy(glo_hbm.at[pl.ds(s * RPT, RPT)],
                      table.at[pl.ds(s * RPT, RPT)])

    @pl.when(jnp.logical_and(c == 0, s == NS - 1))
    def _():
      pltpu.sync_copy(glo_hbm.at[pl.ds(9600, 400)],
                      table.at[pl.ds(9600, 400)])

    @pl.when(jnp.logical_and(c == 1, s < NS - 1))
    def _():
      pltpu.sync_copy(ghi_hbm.at[pl.ds(s * RPT, RPT)],
                      table.at[pl.ds(s * RPT, RPT)])

    @pl.when(jnp.logical_and(c == 1, s == NS - 1))
    def _():
      pltpu.sync_copy(ghi_hbm.at[pl.ds(9600, 400)],
                      table.at[pl.ds(9600, 400)])

    plsc.subcore_barrier()

    for q in range(nq):
      pltpu.sync_copy(src_hbm.at[s, pl.ds(q * kq, kq)], src_q)
      pltpu.sync_copy(dst_hbm.at[s, pl.ds(q * kq, kq)], dst_q)
      for i in range(nbuf - 1):
        pltpu.async_copy(table.at[src_q.at[i]], bufs[i], sems[i])

      def group(p, carry):
        for i in range(nbuf):
          k = nbuf * p + i
          nxt = k + nbuf - 1
          bn = (i + nbuf - 1) % nbuf

          @pl.when(nxt < kq)
          def _(nxt=nxt, bn=bn):
            pltpu.async_copy(table.at[src_q.at[nxt]], bufs[bn], sems[bn])

          pltpu.make_async_copy(table.at[src_q.at[k]], bufs[i],
                                sems[i]).wait()
          pltpu.sync_copy(bufs[i], acc.at[dst_q.at[k]], add=True)
        return carry

      lax.fori_loop(0, kq // nbuf, group, 0)

    plsc.subcore_barrier()

    pltpu.sync_copy(acc.at[pl.ds(row0, RPT)],
                    agg_hbm.at[pl.ds(row0, RPT), pl.ds(c * 64, 64)])

  return pl.kernel(
      body,
      out_type=[jax.ShapeDtypeStruct((NPAD, D_HID), jnp.float32)],
      mesh=_MESH,
      compiler_params=pltpu.CompilerParams(use_tc_tiling_on_sc=False),
      scratch_types=(
          [pltpu.VMEM((kq, chunk), jnp.int32),
           pltpu.VMEM((kq, chunk), jnp.int32)]
          + [pltpu.VMEM((chunk, 64), jnp.float32)] * nbuf
          + [pltpu.VMEM((ZCH, 64), jnp.float32),
             pltpu.VMEM_SHARED((NPAD, 64), jnp.float32),
             pltpu.VMEM_SHARED((NPAD, 64), jnp.float32)]
          + [pltpu.SemaphoreType.DMA] * nbuf),
  )


_seg_spmem = _seg_sum_spmem()

_seg16 = _seg_sum_sc(D_OUT, tc_tiling=False, kq=80, nbuf=4)
_count = _count_sc()

_ROWB = 2000


def _matmul_tc(x, w):
  dn = w.shape[1]

  def body(x_ref, w_ref, o_ref):
    o_ref[...] = jnp.dot(x_ref[...], w_ref[...],
                         preferred_element_type=jnp.float32)

  return pl.pallas_call(
      body,
      grid=(N_NODES // _ROWB,),
      in_specs=[
          pl.BlockSpec((_ROWB, x.shape[1]), lambda i: (i, 0)),
          pl.BlockSpec(w.shape, lambda i: (0, 0)),
      ],
      out_specs=pl.BlockSpec((_ROWB, dn), lambda i: (i, 0)),
      out_shape=jax.ShapeDtypeStruct((N_NODES, dn), jnp.float32),
  )(x, w)


def _matmul_split(x, w):
  def body(x_ref, w_ref, lo_ref, hi_ref):
    g = jnp.dot(x_ref[...], w_ref[...], preferred_element_type=jnp.float32)
    lo_ref[...] = g[:, :64]
    hi_ref[...] = g[:, 64:]

  return pl.pallas_call(
      body,
      grid=(N_NODES // _ROWB,),
      in_specs=[
          pl.BlockSpec((_ROWB, x.shape[1]), lambda i: (i, 0)),
          pl.BlockSpec(w.shape, lambda i: (0, 0)),
      ],
      out_specs=[pl.BlockSpec((_ROWB, 64), lambda i: (i, 0)),
                 pl.BlockSpec((_ROWB, 64), lambda i: (i, 0))],
      out_shape=[jax.ShapeDtypeStruct((N_NODES, 64), jnp.float32),
                 jax.ShapeDtypeStruct((N_NODES, 64), jnp.float32)],
  )(x, w)


def _combine_tc(h, agg_p, cnt_p, w_self, b, apply_relu, w_neigh_next,
                agg_single=False, split_next=False):
  ds = w_self.shape[1]
  have_next = w_neigh_next is not None
  out_shape = [jax.ShapeDtypeStruct((N_NODES, ds), jnp.float32)]
  if have_next:
    dn = w_neigh_next.shape[1]
    out_shape.append(jax.ShapeDtypeStruct((N_NODES, dn), jnp.float32))
  b2 = b.reshape(1, ds)

  def body(h_ref, agg_ref, cnt_ref, ws_ref, b_ref, *rest):
    if have_next:
      if split_next:
        wn_ref, o_ref, glo_ref, ghi_ref = rest
      else:
        wn_ref, o_ref, g_ref = rest
    else:
      (o_ref,) = rest
    agg = agg_ref[...] if agg_single else agg_ref[0] + agg_ref[1]
    deg = cnt_ref[0][:, :1] + cnt_ref[1][:, :1]
    h_neigh = agg / jnp.maximum(deg, 1.0)
    o = (jnp.dot(h_ref[...], ws_ref[...], preferred_element_type=jnp.float32)
         + h_neigh + b_ref[...])
    if apply_relu:
      o = jnp.maximum(o, 0.0)
    o_ref[...] = o
    if have_next:
      g = jnp.dot(o, wn_ref[...], preferred_element_type=jnp.float32)
      if split_next:
        glo_ref[...] = g[:, :64]
        ghi_ref[...] = g[:, 64:]
      else:
        g_ref[...] = g

  in_specs = [
      pl.BlockSpec((_ROWB, h.shape[1]), lambda i: (i, 0)),
      (pl.BlockSpec((_ROWB, ds), lambda i: (i, 0)) if agg_single
       else pl.BlockSpec((NC, _ROWB, ds), lambda i: (0, i, 0))),
      pl.BlockSpec((NC, _ROWB, 16), lambda i: (0, i, 0)),
      pl.BlockSpec(w_self.shape, lambda i: (0, 0)),
      pl.BlockSpec((1, ds), lambda i: (0, 0)),
  ]
  out_specs = [pl.BlockSpec((_ROWB, ds), lambda i: (i, 0))]
  args = [h, agg_p, cnt_p, w_self, b2]
  if have_next:
    in_specs.append(pl.BlockSpec(w_neigh_next.shape, lambda i: (0, 0)))
    if split_next:
      out_specs += [pl.BlockSpec((_ROWB, 64), lambda i: (i, 0)),
                    pl.BlockSpec((_ROWB, 64), lambda i: (i, 0))]
      out_shape[1:] = [jax.ShapeDtypeStruct((N_NODES, 64), jnp.float32),
                       jax.ShapeDtypeStruct((N_NODES, 64), jnp.float32)]
    else:
      out_specs.append(pl.BlockSpec((_ROWB, dn), lambda i: (i, 0)))
    args.append(w_neigh_next)

  res = pl.pallas_call(
      body,
      grid=(N_NODES // _ROWB,),
      in_specs=in_specs,
      out_specs=out_specs,
      out_shape=out_shape,
  )(*args)
  return res if have_next else res[0]


def kernel(x, edge_index, Wself0, Wneigh0, b0, Wself1, Wneigh1, b1,
           Wself2, Wneigh2, b2):
  ei = edge_index.astype(jnp.int32)
  srcf = jnp.concatenate([ei[0], _PAD_SRC])
  dstf = jnp.concatenate([ei[1], _PAD_DST])
  src2 = srcf.reshape(NW, KCH, CHUNK)
  dst2 = dstf.reshape(NW, KCH, CHUNK)
  srcS = srcf.reshape(NS, EPAD // NS // 64, 64)
  dstS = dstf.reshape(NS, EPAD // NS // 64, 64)

  (cnt,) = _count(dst2)
  g0lo, g0hi = _matmul_split(x, Wneigh0)
  (agg0,) = _seg_spmem(g0lo, g0hi, srcS, dstS)
  h1, g1lo, g1hi = _combine_tc(x, agg0, cnt, Wself0, b0, True, Wneigh1,
                               agg_single=True, split_next=True)
  (agg1,) = _seg_spmem(g1lo, g1hi, srcS, dstS)
  h2, g2 = _combine_tc(h1, agg1, cnt, Wself1, b1, True, Wneigh2,
                       agg_single=True)
  (agg2,) = _seg16(g2, src2, dst2)
  out = _combine_tc(h2, agg2, cnt, Wself2, b2, False, None)
  return out


# speedup vs baseline: 1.3216x; 1.3216x over previous
import jax
import jax.numpy as jnp
import numpy as np
from jax import lax
from jax.experimental import pallas as pl
from jax.experimental.pallas import tpu as pltpu
from jax.experimental.pallas import tpu_sc as plsc

N_NODES = 10000
N_EDGES = 320000
D_IN = 128
D_HID = 128
D_OUT = 16

NC = 2
NS = 16
NW = NC * NS
CHUNK = 128
EPAD = 327680
EPT = EPAD // NW
KCH = EPT // CHUNK
KQ = 40
NQ = KCH // KQ
NPAD = 10240
RPT = NPAD // NS
ZCH = 40
NZ = RPT // ZCH

_MESH = plsc.VectorSubcoreMesh(core_axis_name="c", subcore_axis_name="s")

_NPADE = EPAD - N_EDGES
_PAD_SRC = np.arange(_NPADE, dtype=np.int32) % N_NODES
_PAD_DST = (N_NODES + np.arange(_NPADE, dtype=np.int32) % (NPAD - N_NODES)).astype(np.int32)


def _zero_rows(ref, nrows, width):
  zf = jnp.zeros((16,), jnp.float32)

  def zrow(r, carry):
    for j in range(width // 16):
      ref[r, pl.ds(16 * j, 16)] = zf
    return carry

  lax.fori_loop(0, nrows, zrow, 0)


def _seg_sum_sc(width, tc_tiling=True, kq=40, nbuf=2, chunk=CHUNK):
  kch = EPT // chunk
  nq = kch // kq

  def body(g_hbm, src_hbm, dst_hbm, agg_hbm, src_q, dst_q, *rest):
    bufs = rest[:nbuf]
    zbuf = rest[nbuf]
    acc = rest[nbuf + 1]
    sems = rest[nbuf + 2:]
    c = lax.axis_index("c")
    s = lax.axis_index("s")
    wid = c * NS + s
    row0 = s * RPT

    _zero_rows(zbuf, ZCH, width)
    for t in range(NZ):
      pltpu.sync_copy(zbuf, acc.at[pl.ds(row0 + t * ZCH, ZCH)])

    plsc.subcore_barrier()

    for q in range(nq):
      pltpu.sync_copy(src_hbm.at[wid, pl.ds(q * kq, kq)], src_q)
      pltpu.sync_copy(dst_hbm.at[wid, pl.ds(q * kq, kq)], dst_q)
      for i in range(nbuf - 1):
        pltpu.async_copy(g_hbm.at[src_q.at[i]], bufs[i], sems[i])

      def group(p, carry):
        for i in range(nbuf):
          k = nbuf * p + i
          nxt = k + nbuf - 1
          b = i
          bn = (i + nbuf - 1) % nbuf

          @pl.when(nxt < kq)
          def _(nxt=nxt, bn=bn):
            pltpu.async_copy(g_hbm.at[src_q.at[nxt]], bufs[bn], sems[bn])

          pltpu.make_async_copy(g_hbm.at[src_q.at[k]], bufs[b],
                                sems[b]).wait()
          pltpu.sync_copy(bufs[b], acc.at[dst_q.at[k]], add=True)
        return carry

      lax.fori_loop(0, kq // nbuf, group, 0)

    plsc.subcore_barrier()

    pltpu.sync_copy(acc.at[pl.ds(row0, RPT)], agg_hbm.at[c, pl.ds(row0, RPT)])

  return pl.kernel(
      body,
      out_type=[jax.ShapeDtypeStruct((NC, NPAD, width), jnp.float32)],
      mesh=_MESH,
      compiler_params=pltpu.CompilerParams(use_tc_tiling_on_sc=tc_tiling),
      scratch_types=(
          [pltpu.VMEM((kq, chunk), jnp.int32),
           pltpu.VMEM((kq, chunk), jnp.int32)]
          + [pltpu.VMEM((chunk, width), jnp.float32)] * nbuf
          + [pltpu.VMEM((ZCH, width), jnp.float32),
             pltpu.VMEM_SHARED((NPAD, width), jnp.float32)]
          + [pltpu.SemaphoreType.DMA] * nbuf),
  )


def _count_sc():
  kq = 80

  def body(dst_hbm, cnt_hbm, dst_q, ones_v, zcnt, cacc, sem):
    c = lax.axis_index("c")
    s = lax.axis_index("s")
    wid = c * NS + s
    row0 = s * RPT

    _zero_rows(zcnt, ZCH, 16)
    for t in range(NZ):
      pltpu.sync_copy(zcnt, cacc.at[pl.ds(row0 + t * ZCH, ZCH)])

    onef = jnp.ones((16,), jnp.float32)

    def orow(r, carry):
      ones_v[r, :] = onef
      return carry

    lax.fori_loop(0, CHUNK, orow, 0)

    plsc.subcore_barrier()

    pltpu.sync_copy(dst_hbm.at[wid], dst_q)

    def group(p, carry):
      for i in range(8):
        pltpu.async_copy(ones_v, cacc.at[dst_q.at[8 * p + i]], sem,
                         add=True)
      for i in range(8):
        pltpu.make_async_copy(ones_v, cacc.at[dst_q.at[8 * p + i]],
                              sem).wait()
      return carry

    lax.fori_loop(0, kq // 8, group, 0)

    plsc.subcore_barrier()

    pltpu.sync_copy(cacc.at[pl.ds(row0, RPT)], cnt_hbm.at[c, pl.ds(row0, RPT)])

  return pl.kernel(
      body,
      out_type=[jax.ShapeDtypeStruct((NC, NPAD, 16), jnp.float32)],
      mesh=_MESH,
      compiler_params=pltpu.CompilerParams(use_tc_tiling_on_sc=False),
      scratch_types=[
          pltpu.VMEM((kq, CHUNK), jnp.int32),
          pltpu.VMEM((CHUNK, 16), jnp.float32),
          pltpu.VMEM((ZCH, 16), jnp.float32),
          pltpu.VMEM_SHARED((NPAD, 16), jnp.float32),
          pltpu.SemaphoreType.DMA,
      ],
  )


_seg128 = _seg_sum_sc(D_HID, tc_tiling=True, kq=40, nbuf=4, chunk=64)
_seg16 = _seg_sum_sc(D_OUT, tc_tiling=False, kq=80, nbuf=4)
_count = _count_sc()

_ROWB = 2000


def _matmul_tc(x, w):
  dn = w.shape[1]

  def body(x_ref, w_ref, o_ref):
    o_ref[...] = jnp.dot(x_ref[...], w_ref[...],
                         preferred_element_type=jnp.float32)

  return pl.pallas_call(
      body,
      grid=(N_NODES // _ROWB,),
      in_specs=[
          pl.BlockSpec((_ROWB, x.shape[1]), lambda i: (i, 0)),
          pl.BlockSpec(w.shape, lambda i: (0, 0)),
      ],
      out_specs=pl.BlockSpec((_ROWB, dn), lambda i: (i, 0)),
      out_shape=jax.ShapeDtypeStruct((N_NODES, dn), jnp.float32),
  )(x, w)


def _combine_tc(h, agg_p, cnt_p, w_self, b, apply_relu, w_neigh_next):
  ds = w_self.shape[1]
  have_next = w_neigh_next is not None
  out_shape = [jax.ShapeDtypeStruct((N_NODES, ds), jnp.float32)]
  if have_next:
    dn = w_neigh_next.shape[1]
    out_shape.append(jax.ShapeDtypeStruct((N_NODES, dn), jnp.float32))
  b2 = b.reshape(1, ds)

  def body(h_ref, agg_ref, cnt_ref, ws_ref, b_ref, *rest):
    if have_next:
      wn_ref, o_ref, g_ref = rest
    else:
      (o_ref,) = rest
    agg = agg_ref[0] + agg_ref[1]
    deg = cnt_ref[0][:, :1] + cnt_ref[1][:, :1]
    h_neigh = agg / jnp.maximum(deg, 1.0)
    o = (jnp.dot(h_ref[...], ws_ref[...], preferred_element_type=jnp.float32)
         + h_neigh + b_ref[...])
    if apply_relu:
      o = jnp.maximum(o, 0.0)
    o_ref[...] = o
    if have_next:
      g_ref[...] = jnp.dot(o, wn_ref[...],
                           preferred_element_type=jnp.float32)

  in_specs = [
      pl.BlockSpec((_ROWB, h.shape[1]), lambda i: (i, 0)),
      pl.BlockSpec((NC, _ROWB, ds), lambda i: (0, i, 0)),
      pl.BlockSpec((NC, _ROWB, 16), lambda i: (0, i, 0)),
      pl.BlockSpec(w_self.shape, lambda i: (0, 0)),
      pl.BlockSpec((1, ds), lambda i: (0, 0)),
  ]
  out_specs = [pl.BlockSpec((_ROWB, ds), lambda i: (i, 0))]
  args = [h, agg_p, cnt_p, w_self, b2]
  if have_next:
    in_specs.append(pl.BlockSpec(w_neigh_next.shape, lambda i: (0, 0)))
    out_specs.append(pl.BlockSpec((_ROWB, dn), lambda i: (i, 0)))
    args.append(w_neigh_next)

  res = pl.pallas_call(
      body,
      grid=(N_NODES // _ROWB,),
      in_specs=in_specs,
      out_specs=out_specs,
      out_shape=out_shape,
  )(*args)
  return res if have_next else res[0]


def kernel(x, edge_index, Wself0, Wneigh0, b0, Wself1, Wneigh1, b1,
           Wself2, Wneigh2, b2):
  ei = edge_index.astype(jnp.int32)
  srcf = jnp.concatenate([ei[0], _PAD_SRC])
  dstf = jnp.concatenate([ei[1], _PAD_DST])
  src2 = srcf.reshape(NW, KCH, CHUNK)
  dst2 = dstf.reshape(NW, KCH, CHUNK)
  src2a = srcf.reshape(NW, EPT // 64, 64)
  dst2a = dstf.reshape(NW, EPT // 64, 64)

  (cnt,) = _count(dst2)
  g0 = _matmul_tc(x, Wneigh0)
  (agg0,) = _seg128(g0, src2a, dst2a)
  h1, g1 = _combine_tc(x, agg0, cnt, Wself0, b0, True, Wneigh1)
  (agg1,) = _seg128(g1, src2a, dst2a)
  h2, g2 = _combine_tc(h1, agg1, cnt, Wself1, b1, True, Wneigh2)
  (agg2,) = _seg16(g2, src2, dst2)
  out = _combine_tc(h2, agg2, cnt, Wself2, b2, False, None)
  return out
